# Initial kernel scaffold; baseline (speedup 1.0000x reference)
import functools

import jax
import jax.numpy as jnp
from jax import lax
from jax.experimental import pallas as pl
from jax.experimental.pallas import tpu as pltpu
from jax.experimental.pallas import tpu_sc as plsc

B = 16384
D = 64
NEGK = 5
C = 128
LANES = 16


def _sc_scores(pos_u, pos_v, neg_flat, wu, wv):
    info = plsc.get_sparse_core_info()
    nw = info.num_cores * info.num_subcores
    epw = B // nw
    nchunk = epw // C
    mesh = plsc.VectorSubcoreMesh(core_axis_name="c", subcore_axis_name="s")

    @functools.partial(
        pl.kernel,
        out_type=[jax.ShapeDtypeStruct((B,), jnp.float32),
                  jax.ShapeDtypeStruct((B * NEGK,), jnp.float32)],
        mesh=mesh,
        scratch_types=[
            pltpu.VMEM((C,), jnp.int32),
            pltpu.VMEM((C,), jnp.int32),
            pltpu.VMEM((C * NEGK,), jnp.int32),
            pltpu.VMEM((C, D), jnp.float32),
            pltpu.VMEM((C, D), jnp.float32),
            pltpu.VMEM((C * NEGK, D), jnp.float32),
            pltpu.VMEM((C,), jnp.float32),
            pltpu.VMEM((C * NEGK,), jnp.float32),
            pltpu.SemaphoreType.DMA,
        ],
    )
    def scores(pos_u_hbm, pos_v_hbm, neg_hbm, wu_hbm, wv_hbm,
               pos_out, neg_out, iu, iv, ineg, ru, rv, rn, sp, sn, sem):
        wid = lax.axis_index("s") * info.num_cores + lax.axis_index("c")
        lane = jnp.arange(LANES, dtype=jnp.int32)
        for chunk in range(nchunk):
            b0 = wid * epw + chunk * C
            pltpu.sync_copy(pos_u_hbm.at[pl.ds(b0, C)], iu)
            pltpu.sync_copy(pos_v_hbm.at[pl.ds(b0, C)], iv)
            pltpu.sync_copy(neg_hbm.at[pl.ds(b0 * NEGK, C * NEGK)], ineg)
            cp_u = pltpu.async_copy(wu_hbm.at[iu], ru, sem)
            cp_v = pltpu.async_copy(wv_hbm.at[iv], rv, sem)
            cp_n = pltpu.async_copy(wv_hbm.at[ineg], rn, sem)
            cp_u.wait()
            cp_v.wait()
            cp_n.wait()
            for g in range(C // LANES):
                rowu = lane + (g * LANES)

                def dbody(d, accs, rowu=rowu):
                    col = jnp.full((LANES,), d, jnp.int32)
                    xu = plsc.load_gather(ru, [rowu, col])
                    xv = plsc.load_gather(rv, [rowu, col])
                    out = [accs[0] + xu * xv]
                    for n in range(NEGK):
                        xn = plsc.load_gather(rn, [rowu * NEGK + n, col])
                        out.append(accs[1 + n] + xn * xu)
                    return tuple(out)

                z = jnp.zeros((LANES,), jnp.float32)
                accs = lax.fori_loop(0, D, dbody, (z,) * (1 + NEGK))
                sp[pl.ds(g * LANES, LANES)] = accs[0]
                for n in range(NEGK):
                    plsc.store_scatter(sn, [rowu * NEGK + n], accs[1 + n])
            pltpu.sync_copy(sp, pos_out.at[pl.ds(b0, C)])
            pltpu.sync_copy(sn, neg_out.at[pl.ds(b0 * NEGK, C * NEGK)])

    return scores(pos_u, pos_v, neg_flat, wu, wv)


def _loss(pos_s, neg_s):
    pos2 = pos_s.reshape(B // 128, 128)
    neg2 = neg_s.reshape(B * NEGK // 128, 128)

    def body(p_ref, n_ref, o_ref):
        p = jnp.clip(p_ref[...], -6.0, 6.0)
        n = jnp.clip(n_ref[...], -6.0, 6.0)
        lp = jnp.log1p(jnp.exp(-p))
        ln = jnp.log1p(jnp.exp(n))
        o_ref[0, 0] = (jnp.sum(lp) + jnp.sum(ln)) * (1.0 / B)

    out = pl.pallas_call(
        body,
        out_shape=jax.ShapeDtypeStruct((1, 1), jnp.float32),
    )(pos2, neg2)
    return out[0, 0]


def kernel(pos_u, pos_v, neg_v, snd_u_weight, snd_v_weight):
    pos_s, neg_s = _sc_scores(pos_u, pos_v, neg_v.reshape(-1),
                              snd_u_weight, snd_v_weight)
    return _loss(pos_s, neg_s)


# speedup vs baseline: 1.5829x; 1.5829x over previous
import functools

import jax
import jax.numpy as jnp
from jax import lax
from jax.experimental import pallas as pl
from jax.experimental.pallas import tpu as pltpu
from jax.experimental.pallas import tpu_sc as plsc

B = 16384
D = 64
NEGK = 5
C = 128
LANES = 16


def _sc_scores(pos_u, pos_v, neg_flat, wu, wv):
    info = plsc.get_sparse_core_info()
    nw = info.num_cores * info.num_subcores
    epw = B // nw
    nchunk = epw // C
    mesh = plsc.VectorSubcoreMesh(core_axis_name="c", subcore_axis_name="s")

    @functools.partial(
        pl.kernel,
        out_type=[jax.ShapeDtypeStruct((B,), jnp.float32),
                  jax.ShapeDtypeStruct((B * NEGK,), jnp.float32)],
        mesh=mesh,
        scratch_types=[
            pltpu.VMEM((C,), jnp.int32),
            pltpu.VMEM((C,), jnp.int32),
            pltpu.VMEM((C * NEGK,), jnp.int32),
            pltpu.VMEM((C, D), jnp.float32),
            pltpu.VMEM((C, D), jnp.float32),
            pltpu.VMEM((C * NEGK, D), jnp.float32),
            pltpu.VMEM((C,), jnp.float32),
            pltpu.VMEM((C * NEGK,), jnp.float32),
            pltpu.SemaphoreType.DMA,
        ],
        compiler_params=pltpu.CompilerParams(needs_layout_passes=False,
                                             use_tc_tiling_on_sc=False),
    )
    def scores(pos_u_hbm, pos_v_hbm, neg_hbm, wu_hbm, wv_hbm,
               pos_out, neg_out, iu, iv, ineg, ru, rv, rn, sp, sn, sem):
        wid = lax.axis_index("s") * info.num_cores + lax.axis_index("c")
        lane = jnp.arange(LANES, dtype=jnp.int32)
        for chunk in range(nchunk):
            b0 = wid * epw + chunk * C
            pltpu.sync_copy(pos_u_hbm.at[pl.ds(b0, C)], iu)
            pltpu.sync_copy(pos_v_hbm.at[pl.ds(b0, C)], iv)
            pltpu.sync_copy(neg_hbm.at[pl.ds(b0 * NEGK, C * NEGK)], ineg)
            cp_u = pltpu.async_copy(wu_hbm.at[iu], ru, sem)
            cp_v = pltpu.async_copy(wv_hbm.at[iv], rv, sem)
            cp_n = pltpu.async_copy(wv_hbm.at[ineg], rn, sem)
            cp_u.wait()
            cp_v.wait()
            cp_n.wait()
            for g in range(C // LANES):
                rowu = lane + (g * LANES)
                rown = [rowu * NEGK + n for n in range(NEGK)]

                def dbody(d, accs, rowu=rowu, rown=rown):
                    dcol = jnp.full((LANES,), d, jnp.int32)
                    xu = plsc.load_gather(ru, [rowu, dcol])
                    xv = plsc.load_gather(rv, [rowu, dcol])
                    out = [accs[0] + xu * xv]
                    for n in range(NEGK):
                        xn = plsc.load_gather(rn, [rown[n], dcol])
                        out.append(accs[1 + n] + xn * xu)
                    return tuple(out)

                z = jnp.zeros((LANES,), jnp.float32)
                accs = lax.fori_loop(0, D, dbody, (z,) * (1 + NEGK))
                sp[pl.ds(g * LANES, LANES)] = accs[0]
                for n in range(NEGK):
                    plsc.store_scatter(sn, [rowu * NEGK + n], accs[1 + n])
            pltpu.sync_copy(sp, pos_out.at[pl.ds(b0, C)])
            pltpu.sync_copy(sn, neg_out.at[pl.ds(b0 * NEGK, C * NEGK)])

    return scores(pos_u, pos_v, neg_flat, wu, wv)


def _loss(pos_s, neg_s):
    pos2 = pos_s.reshape(B // 128, 128)
    neg2 = neg_s.reshape(B * NEGK // 128, 128)

    def body(p_ref, n_ref, o_ref):
        p = jnp.clip(p_ref[...], -6.0, 6.0)
        n = jnp.clip(n_ref[...], -6.0, 6.0)
        lp = jnp.log1p(jnp.exp(-p))
        ln = jnp.log1p(jnp.exp(n))
        o_ref[0, 0] = (jnp.sum(lp) + jnp.sum(ln)) * (1.0 / B)

    out = pl.pallas_call(
        body,
        out_shape=jax.ShapeDtypeStruct((1, 1), jnp.float32),
        out_specs=pl.BlockSpec(memory_space=pltpu.SMEM),
    )(pos2, neg2)
    return out[0, 0]


def kernel(pos_u, pos_v, neg_v, snd_u_weight, snd_v_weight):
    pos_s, neg_s = _sc_scores(pos_u, pos_v, neg_v.reshape(-1),
                              snd_u_weight, snd_v_weight)
    return _loss(pos_s, neg_s)
